# baseline (speedup 1.0000x reference)
import functools
import jax
import jax.numpy as jnp
from jax import lax
from jax.experimental import pallas as pl
from jax.experimental.pallas import tpu as pltpu, tpu_sc as plsc

N_NODES = 100000
DEPTH = 16
L = 16
NC, NS = 2, 16
NW = NC * NS

NPAD = 100352
K = 784
N_CH = NPAD // (NS * K)

B, C = 2, 3
HW = 512 * 512
PIX_PER_TILE = HW // NW
PIXSUB = 1024
N_PIX = PIX_PER_TILE // PIXSUB

_MESH = plsc.VectorSubcoreMesh(
    core_axis_name="c", subcore_axis_name="s", num_cores=NC, num_subcores=NS
)
_PARAMS = pltpu.CompilerParams(
    needs_layout_passes=False, use_tc_tiling_on_sc=False)


def _fused_body(attr, res, wsb, anc, p2n, out,
                big, attr_sub, res_sub, wsb_v, anc_blk, nodeval, p2n_blk,
                y_blk, spmem):
    cid = lax.axis_index("c")
    sid = lax.axis_index("s")
    wid = sid * NC + cid
    iota = lax.iota(jnp.int32, L)
    cols = [jnp.full((L,), d, jnp.int32) for d in range(DEPTH)]

    pltpu.sync_copy(wsb, wsb_v)
    w0 = wsb_v[0]
    w1 = wsb_v[1]
    w2 = wsb_v[2]
    bv = wsb_v[3]
    for k in range(N_CH):
        start = jnp.minimum((sid * N_CH + k) * K, N_NODES - K)
        pltpu.sync_copy(attr.at[pl.ds(start, K), :], attr_sub)
        pltpu.sync_copy(res.at[pl.ds(start, K)], res_sub)

        def p1_body(g, _):
            off = g * L
            row = iota + off
            a0 = plsc.load_gather(attr_sub, [row, cols[0]])
            a1 = plsc.load_gather(attr_sub, [row, cols[1]])
            a2 = plsc.load_gather(attr_sub, [row, cols[2]])
            r = res_sub[pl.ds(off, L)]
            logit = a0 * w0 + a1 * w1 + a2 * w2 + bv
            score = 1.0 / (1.0 + jnp.exp(-logit))
            big[pl.ds(start + off, L)] = score * r
            return 0

        lax.fori_loop(0, K // L, p1_body, 0)

    own = pl.ds(sid * (NPAD // NS), NPAD // NS)
    pltpu.sync_copy(big.at[own], spmem.at[own])
    plsc.subcore_barrier()
    pltpu.sync_copy(spmem, big)
    plsc.subcore_barrier()

    for k in range(N_CH):
        start = jnp.minimum((sid * N_CH + k) * K, N_NODES - K)
        pltpu.sync_copy(anc.at[pl.ds(start, K), :], anc_blk)

        def p2_body(g, _):
            off = g * L
            row = iota + off
            idx = plsc.load_gather(anc_blk, [row, cols[0]])
            acc = plsc.load_gather(big, [idx])
            for d in range(1, DEPTH):
                idx = plsc.load_gather(anc_blk, [row, cols[d]])
                acc = acc + plsc.load_gather(big, [idx])
            nodeval[pl.ds(off, L)] = acc
            return 0

        lax.fori_loop(0, K // L, p2_body, 0)
        pltpu.sync_copy(nodeval, spmem.at[pl.ds(start, K)])

    plsc.subcore_barrier()
    pltpu.sync_copy(spmem, big)

    for q in range(N_PIX):
        poff = wid * PIX_PER_TILE + q * PIXSUB
        pltpu.sync_copy(p2n.at[pl.ds(poff, PIXSUB)], p2n_blk)

        def p3_body(g, _):
            off = g * L
            val = plsc.load_gather(big, [p2n_blk[pl.ds(off, L)]])
            y_blk[off // 512, pl.ds(off % 512, L)] = val
            return 0

        lax.fori_loop(0, PIXSUB // L, p3_body, 0)
        row0 = wid * (PIX_PER_TILE // 512) + q * (PIXSUB // 512)
        for b in range(B):
            for c in range(C):
                pltpu.sync_copy(
                    y_blk, out.at[b, c, pl.ds(row0, PIXSUB // 512), :])


_fused_kernel = functools.partial(
    pl.kernel,
    out_type=jax.ShapeDtypeStruct((B, C, 512, 512), jnp.float32),
    mesh=_MESH,
    scratch_types=[
        pltpu.VMEM((NPAD,), jnp.float32),
        pltpu.VMEM((K, 3), jnp.float32),
        pltpu.VMEM((K,), jnp.float32),
        pltpu.VMEM((4, L), jnp.float32),
        pltpu.VMEM((K, DEPTH), jnp.int32),
        pltpu.VMEM((K,), jnp.float32),
        pltpu.VMEM((PIXSUB,), jnp.int32),
        pltpu.VMEM((PIXSUB // 512, 512), jnp.float32),
        pltpu.VMEM_SHARED((NPAD,), jnp.float32),
    ],
    compiler_params=_PARAMS,
)(_fused_body)


@jax.jit
def kernel(x, attributes, residues, weight, bias, pixel_to_node, ancestors):
    Bn, Cn, Hn, Wn = x.shape
    wsb = jnp.concatenate(
        [jnp.broadcast_to(weight[:, None], (3, L)),
         jnp.broadcast_to(bias[:, None], (1, L))], axis=0)
    return _fused_kernel(
        attributes, residues, wsb,
        ancestors.astype(jnp.int32), pixel_to_node.astype(jnp.int32))


# speedup vs baseline: 2.4636x; 2.4636x over previous
import functools
import jax
import jax.numpy as jnp
from jax import lax
from jax.experimental import pallas as pl
from jax.experimental.pallas import tpu as pltpu, tpu_sc as plsc

N_NODES = 100000
DEPTH = 16
L = 16
NC, NS = 2, 16
NW = NC * NS

NPAD = 100352
K = 784
N_CH = NPAD // (NS * K)
LAST = N_NODES - K

B, C = 2, 3
HW = 512 * 512
PIX_PER_TILE = HW // NW
PIXSUB = 1024
N_PIX = PIX_PER_TILE // PIXSUB

_MESH = plsc.VectorSubcoreMesh(
    core_axis_name="c", subcore_axis_name="s", num_cores=NC, num_subcores=NS
)
_PARAMS = pltpu.CompilerParams(
    needs_layout_passes=False, use_tc_tiling_on_sc=False)


def _fused_body(attr_t, res, wsb, anc_t, p2n, out,
                big, attr_sub, res_sub, wsb_v, anc_blk, nodeval, p2n_blk,
                y_blk, spmem, sem_in, sem_anc, sem_out):
    cid = lax.axis_index("c")
    sid = lax.axis_index("s")
    wid = sid * NC + cid

    def anc_fetch(start):
        return [
            pltpu.async_copy(
                anc_t.at[pl.ds(d * N_NODES + start, K)], anc_blk.at[d], sem_anc)
            for d in range(DEPTH)
        ]

    def chunk_start(k):
        return jnp.minimum((sid * N_CH + k) * K, LAST)

    anc_descs = anc_fetch(chunk_start(0))

    pltpu.sync_copy(wsb, wsb_v)
    w0 = wsb_v[pl.ds(0, L)]
    w1 = wsb_v[pl.ds(L, L)]
    w2 = wsb_v[pl.ds(2 * L, L)]
    bv = wsb_v[pl.ds(3 * L, L)]
    for k in range(N_CH):
        start = chunk_start(k)
        descs = [
            pltpu.async_copy(
                attr_t.at[pl.ds(j * N_NODES + start, K)], attr_sub.at[j],
                sem_in)
            for j in range(3)
        ]
        descs.append(
            pltpu.async_copy(res.at[pl.ds(start, K)], res_sub, sem_in))
        for d_ in descs:
            d_.wait()

        def p1_body(g, _):
            off = g * L
            a0 = attr_sub[0, pl.ds(off, L)]
            a1 = attr_sub[1, pl.ds(off, L)]
            a2 = attr_sub[2, pl.ds(off, L)]
            r = res_sub[pl.ds(off, L)]
            logit = a0 * w0 + a1 * w1 + a2 * w2 + bv
            score = 1.0 / (1.0 + jnp.exp(-logit))
            big[pl.ds(start + off, L)] = score * r
            return 0

        lax.fori_loop(0, K // L, p1_body, 0)

    own = pl.ds(sid * (NPAD // NS), NPAD // NS)
    pltpu.sync_copy(big.at[own], spmem.at[own])
    plsc.subcore_barrier()
    pltpu.sync_copy(spmem, big)
    plsc.subcore_barrier()

    for k in range(N_CH):
        start = chunk_start(k)
        for d_ in anc_descs:
            d_.wait()

        def p2_body(g, _):
            off = g * L
            acc = plsc.load_gather(big, [anc_blk[0, pl.ds(off, L)]])
            for d in range(1, DEPTH):
                acc = acc + plsc.load_gather(big, [anc_blk[d, pl.ds(off, L)]])
            nodeval[pl.ds(off, L)] = acc
            return 0

        lax.fori_loop(0, K // L, p2_body, 0)
        if k + 1 < N_CH:
            anc_descs = anc_fetch(chunk_start(k + 1))
        pltpu.sync_copy(nodeval, spmem.at[pl.ds(start, K)])

    plsc.subcore_barrier()
    pltpu.sync_copy(spmem, big)

    out_descs = [[], []]
    for q in range(N_PIX):
        par = q % 2
        poff = wid * PIX_PER_TILE + q * PIXSUB
        pltpu.sync_copy(p2n.at[pl.ds(poff, PIXSUB)], p2n_blk)
        for d_ in out_descs[par]:
            d_.wait()

        def p3_body(g, _, par=par):
            off = g * L
            y_blk[par, pl.ds(off, L)] = plsc.load_gather(
                big, [p2n_blk[pl.ds(off, L)]])
            return 0

        lax.fori_loop(0, PIXSUB // L, p3_body, 0)
        out_descs[par] = [
            pltpu.async_copy(
                y_blk.at[par], out.at[pl.ds(bc * HW + poff, PIXSUB)], sem_out)
            for bc in range(B * C)
        ]
    for par in range(2):
        for d_ in out_descs[par]:
            d_.wait()


_fused_kernel = functools.partial(
    pl.kernel,
    out_type=jax.ShapeDtypeStruct((B * C * HW,), jnp.float32),
    mesh=_MESH,
    scratch_types=[
        pltpu.VMEM((NPAD,), jnp.float32),
        pltpu.VMEM((3, K), jnp.float32),
        pltpu.VMEM((K,), jnp.float32),
        pltpu.VMEM((4 * L,), jnp.float32),
        pltpu.VMEM((DEPTH, K), jnp.int32),
        pltpu.VMEM((K,), jnp.float32),
        pltpu.VMEM((PIXSUB,), jnp.int32),
        pltpu.VMEM((2, PIXSUB), jnp.float32),
        pltpu.VMEM_SHARED((NPAD,), jnp.float32),
        pltpu.SemaphoreType.DMA,
        pltpu.SemaphoreType.DMA,
        pltpu.SemaphoreType.DMA,
    ],
    compiler_params=_PARAMS,
)(_fused_body)


@jax.jit
def kernel(x, attributes, residues, weight, bias, pixel_to_node, ancestors):
    Bn, Cn, Hn, Wn = x.shape
    attr_t = attributes.T.reshape(-1)
    anc_t = ancestors.astype(jnp.int32).T.reshape(-1)
    wsb = jnp.concatenate(
        [jnp.broadcast_to(weight[:, None], (3, L)),
         jnp.broadcast_to(bias[:, None], (1, L))], axis=0).reshape(-1)
    out_flat = _fused_kernel(
        attr_t, residues, wsb, anc_t, pixel_to_node.astype(jnp.int32))
    return out_flat.reshape(Bn, Cn, Hn, Wn)


# speedup vs baseline: 2.5859x; 1.0497x over previous
import functools
import jax
import jax.numpy as jnp
from jax import lax
from jax.experimental import pallas as pl
from jax.experimental.pallas import tpu as pltpu, tpu_sc as plsc

N_NODES = 100000
DEPTH = 16
L = 16
NC, NS = 2, 16
NW = NC * NS

NPAD = 100352
K1 = 448
N_CH1 = NPAD // (NS * K1)
LAST1 = N_NODES - K1
K2 = 784
N_CH2 = NPAD // (NS * K2)
LAST2 = N_NODES - K2

B, C = 2, 3
HW = 512 * 512
PIX_PER_TILE = HW // NW
PIXSUB = 1024
N_PIX = PIX_PER_TILE // PIXSUB

_MESH = plsc.VectorSubcoreMesh(
    core_axis_name="c", subcore_axis_name="s", num_cores=NC, num_subcores=NS
)
_PARAMS = pltpu.CompilerParams(
    needs_layout_passes=False, use_tc_tiling_on_sc=False)


def _fused_body(attr_t, res, wsb, anc_t, p2n, out,
                big, attr_sub, res_sub, wsb_v, anc_blk, nodeval, p2n_blk,
                y_blk, spmem, sem_in, sem_anc, sem_out):
    cid = lax.axis_index("c")
    sid = lax.axis_index("s")
    wid = sid * NC + cid

    def anc_fetch(start):
        return [
            pltpu.async_copy(
                anc_t.at[pl.ds(d * N_NODES + start, K2)], anc_blk.at[d],
                sem_anc)
            for d in range(DEPTH)
        ]

    def start1(k):
        return jnp.minimum((sid * N_CH1 + k) * K1, LAST1)

    def start2(k):
        return jnp.minimum((sid * N_CH2 + k) * K2, LAST2)

    def attr_fetch(start, par):
        descs = [
            pltpu.async_copy(
                attr_t.at[pl.ds(j * N_NODES + start, K1)],
                attr_sub.at[par, j], sem_in)
            for j in range(3)
        ]
        descs.append(
            pltpu.async_copy(res.at[pl.ds(start, K1)], res_sub.at[par],
                             sem_in))
        return descs

    anc_descs = anc_fetch(start2(0))

    pltpu.sync_copy(wsb, wsb_v)
    w0 = wsb_v[pl.ds(0, L)]
    w1 = wsb_v[pl.ds(L, L)]
    w2 = wsb_v[pl.ds(2 * L, L)]
    bv = wsb_v[pl.ds(3 * L, L)]
    descs = [attr_fetch(start1(0), 0), None]
    for k in range(N_CH1):
        par = k % 2
        start = start1(k)
        for d_ in descs[par]:
            d_.wait()
        if k + 1 < N_CH1:
            descs[1 - par] = attr_fetch(start1(k + 1), 1 - par)

        def p1_body(g, _, par=par):
            off = g * L
            a0 = attr_sub[par, 0, pl.ds(off, L)]
            a1 = attr_sub[par, 1, pl.ds(off, L)]
            a2 = attr_sub[par, 2, pl.ds(off, L)]
            r = res_sub[par, pl.ds(off, L)]
            logit = a0 * w0 + a1 * w1 + a2 * w2 + bv
            score = 1.0 / (1.0 + jnp.exp(-logit))
            big[pl.ds(start + off, L)] = score * r
            return 0

        lax.fori_loop(0, K1 // L, p1_body, 0)

    own = pl.ds(sid * (NPAD // NS), NPAD // NS)
    pltpu.sync_copy(big.at[own], spmem.at[own])
    plsc.subcore_barrier()
    pltpu.sync_copy(spmem, big)
    plsc.subcore_barrier()

    nv_descs = []
    for k in range(N_CH2):
        start = start2(k)
        for d_ in anc_descs:
            d_.wait()

        def p2_body(g, _):
            off = g * L
            acc = plsc.load_gather(big, [anc_blk[0, pl.ds(off, L)]])
            for d in range(1, DEPTH):
                acc = acc + plsc.load_gather(big, [anc_blk[d, pl.ds(off, L)]])
            nodeval[pl.ds(off, L)] = acc
            return 0

        lax.fori_loop(0, K2 // L, p2_body, 0)
        for d_ in nv_descs:
            d_.wait()
        nv_descs = [
            pltpu.async_copy(nodeval, spmem.at[pl.ds(start, K2)], sem_in)]
        if k + 1 < N_CH2:
            anc_descs = anc_fetch(start2(k + 1))
    for d_ in nv_descs:
        d_.wait()

    plsc.subcore_barrier()
    pltpu.sync_copy(spmem, big)

    out_descs = [[], []]
    for q in range(N_PIX):
        par = q % 2
        poff = wid * PIX_PER_TILE + q * PIXSUB
        pltpu.sync_copy(p2n.at[pl.ds(poff, PIXSUB)], p2n_blk)
        for d_ in out_descs[par]:
            d_.wait()

        def p3_body(g, _, par=par):
            off = g * L
            y_blk[par, pl.ds(off, L)] = plsc.load_gather(
                big, [p2n_blk[pl.ds(off, L)]])
            return 0

        lax.fori_loop(0, PIXSUB // L, p3_body, 0)
        out_descs[par] = [
            pltpu.async_copy(
                y_blk.at[par], out.at[pl.ds(bc * HW + poff, PIXSUB)], sem_out)
            for bc in range(B * C)
        ]
    for par in range(2):
        for d_ in out_descs[par]:
            d_.wait()


_fused_kernel = functools.partial(
    pl.kernel,
    out_type=jax.ShapeDtypeStruct((B * C * HW,), jnp.float32),
    mesh=_MESH,
    scratch_types=[
        pltpu.VMEM((NPAD,), jnp.float32),
        pltpu.VMEM((2, 3, K1), jnp.float32),
        pltpu.VMEM((2, K1), jnp.float32),
        pltpu.VMEM((4 * L,), jnp.float32),
        pltpu.VMEM((DEPTH, K2), jnp.int32),
        pltpu.VMEM((K2,), jnp.float32),
        pltpu.VMEM((PIXSUB,), jnp.int32),
        pltpu.VMEM((2, PIXSUB), jnp.float32),
        pltpu.VMEM_SHARED((NPAD,), jnp.float32),
        pltpu.SemaphoreType.DMA,
        pltpu.SemaphoreType.DMA,
        pltpu.SemaphoreType.DMA,
    ],
    compiler_params=_PARAMS,
)(_fused_body)


@jax.jit
def kernel(x, attributes, residues, weight, bias, pixel_to_node, ancestors):
    Bn, Cn, Hn, Wn = x.shape
    attr_t = attributes.T.reshape(-1)
    anc_t = ancestors.astype(jnp.int32).T.reshape(-1)
    wsb = jnp.concatenate(
        [jnp.broadcast_to(weight[:, None], (3, L)),
         jnp.broadcast_to(bias[:, None], (1, L))], axis=0).reshape(-1)
    out_flat = _fused_kernel(
        attr_t, residues, wsb, anc_t, pixel_to_node.astype(jnp.int32))
    return out_flat.reshape(Bn, Cn, Hn, Wn)


# speedup vs baseline: 2.5860x; 1.0000x over previous
import functools
import jax
import jax.numpy as jnp
from jax import lax
from jax.experimental import pallas as pl
from jax.experimental.pallas import tpu as pltpu, tpu_sc as plsc

N_NODES = 100000
DEPTH = 16
L = 16
NC, NS = 2, 16
NW = NC * NS

NPAD = 100352
K1 = 448
N_CH1 = NPAD // (NS * K1)
LAST1 = N_NODES - K1
K2 = 784
N_CH2 = NPAD // (NS * K2)
LAST2 = N_NODES - K2

B, C = 2, 3
HW = 512 * 512
PIX_PER_TILE = HW // NW
PIXSUB = 1024
N_PIX = PIX_PER_TILE // PIXSUB

_MESH = plsc.VectorSubcoreMesh(
    core_axis_name="c", subcore_axis_name="s", num_cores=NC, num_subcores=NS
)
_PARAMS = pltpu.CompilerParams(
    needs_layout_passes=False, use_tc_tiling_on_sc=False)


def _fused_body(attr_t, res, wsb, anc_t, p2n, out,
                big, attr_sub, res_sub, wsb_v, anc_blk, nodeval, p2n_blk,
                y_blk, spmem, sem_in, sem_anc, sem_out):
    cid = lax.axis_index("c")
    sid = lax.axis_index("s")
    wid = sid * NC + cid

    def anc_fetch(start):
        return [
            pltpu.async_copy(
                anc_t.at[pl.ds(d * N_NODES + start, K2)], anc_blk.at[d],
                sem_anc)
            for d in range(DEPTH)
        ]

    def start1(k):
        return jnp.minimum((sid * N_CH1 + k) * K1, LAST1)

    def start2(k):
        return jnp.minimum((sid * N_CH2 + k) * K2, LAST2)

    def attr_fetch(start, par):
        descs = [
            pltpu.async_copy(
                attr_t.at[pl.ds(j * N_NODES + start, K1)],
                attr_sub.at[par, j], sem_in)
            for j in range(3)
        ]
        descs.append(
            pltpu.async_copy(res.at[pl.ds(start, K1)], res_sub.at[par],
                             sem_in))
        return descs

    anc_descs = anc_fetch(start2(0))
    import contextlib
    scope = jax.named_scope

    ph1 = scope("phase1"); ph1.__enter__()
    pltpu.sync_copy(wsb, wsb_v)
    w0 = wsb_v[pl.ds(0, L)]
    w1 = wsb_v[pl.ds(L, L)]
    w2 = wsb_v[pl.ds(2 * L, L)]
    bv = wsb_v[pl.ds(3 * L, L)]
    descs = [attr_fetch(start1(0), 0), None]
    for k in range(N_CH1):
        par = k % 2
        start = start1(k)
        for d_ in descs[par]:
            d_.wait()
        if k + 1 < N_CH1:
            descs[1 - par] = attr_fetch(start1(k + 1), 1 - par)

        def p1_body(g, _, par=par):
            off = g * L
            a0 = attr_sub[par, 0, pl.ds(off, L)]
            a1 = attr_sub[par, 1, pl.ds(off, L)]
            a2 = attr_sub[par, 2, pl.ds(off, L)]
            r = res_sub[par, pl.ds(off, L)]
            logit = a0 * w0 + a1 * w1 + a2 * w2 + bv
            score = 1.0 / (1.0 + jnp.exp(-logit))
            big[pl.ds(start + off, L)] = score * r
            return 0

        lax.fori_loop(0, K1 // L, p1_body, 0)

    ph1.__exit__(None, None, None)
    xch = scope("exchange1"); xch.__enter__()
    own = pl.ds(sid * (NPAD // NS), NPAD // NS)
    pltpu.sync_copy(big.at[own], spmem.at[own])
    plsc.subcore_barrier()
    pltpu.sync_copy(spmem, big)
    plsc.subcore_barrier()

    xch.__exit__(None, None, None)
    ph2 = scope("phase2"); ph2.__enter__()
    nv_descs = []
    for k in range(N_CH2):
        start = start2(k)
        for d_ in anc_descs:
            d_.wait()

        def p2_body(g, _):
            off = g * L
            acc = plsc.load_gather(big, [anc_blk[0, pl.ds(off, L)]])
            for d in range(1, DEPTH):
                acc = acc + plsc.load_gather(big, [anc_blk[d, pl.ds(off, L)]])
            nodeval[pl.ds(off, L)] = acc
            return 0

        lax.fori_loop(0, K2 // L, p2_body, 0)
        for d_ in nv_descs:
            d_.wait()
        nv_descs = [
            pltpu.async_copy(nodeval, spmem.at[pl.ds(start, K2)], sem_in)]
        if k + 1 < N_CH2:
            anc_descs = anc_fetch(start2(k + 1))
    for d_ in nv_descs:
        d_.wait()

    ph2.__exit__(None, None, None)
    xch2 = scope("exchange2"); xch2.__enter__()
    plsc.subcore_barrier()
    pltpu.sync_copy(spmem, big)
    xch2.__exit__(None, None, None)
    ph3 = scope("phase3"); ph3.__enter__()

    out_descs = [[], []]
    for q in range(N_PIX):
        par = q % 2
        poff = wid * PIX_PER_TILE + q * PIXSUB
        pltpu.sync_copy(p2n.at[pl.ds(poff, PIXSUB)], p2n_blk)
        for d_ in out_descs[par]:
            d_.wait()

        def p3_body(g, _, par=par):
            off = g * L
            y_blk[par, pl.ds(off, L)] = plsc.load_gather(
                big, [p2n_blk[pl.ds(off, L)]])
            return 0

        lax.fori_loop(0, PIXSUB // L, p3_body, 0)
        out_descs[par] = [
            pltpu.async_copy(
                y_blk.at[par], out.at[pl.ds(bc * HW + poff, PIXSUB)], sem_out)
            for bc in range(B * C)
        ]
    for par in range(2):
        for d_ in out_descs[par]:
            d_.wait()
    ph3.__exit__(None, None, None)


_fused_kernel = functools.partial(
    pl.kernel,
    out_type=jax.ShapeDtypeStruct((B * C * HW,), jnp.float32),
    mesh=_MESH,
    scratch_types=[
        pltpu.VMEM((NPAD,), jnp.float32),
        pltpu.VMEM((2, 3, K1), jnp.float32),
        pltpu.VMEM((2, K1), jnp.float32),
        pltpu.VMEM((4 * L,), jnp.float32),
        pltpu.VMEM((DEPTH, K2), jnp.int32),
        pltpu.VMEM((K2,), jnp.float32),
        pltpu.VMEM((PIXSUB,), jnp.int32),
        pltpu.VMEM((2, PIXSUB), jnp.float32),
        pltpu.VMEM_SHARED((NPAD,), jnp.float32),
        pltpu.SemaphoreType.DMA,
        pltpu.SemaphoreType.DMA,
        pltpu.SemaphoreType.DMA,
    ],
    compiler_params=_PARAMS,
)(_fused_body)


@jax.jit
def kernel(x, attributes, residues, weight, bias, pixel_to_node, ancestors):
    Bn, Cn, Hn, Wn = x.shape
    attr_t = attributes.T.reshape(-1)
    anc_t = ancestors.astype(jnp.int32).T.reshape(-1)
    wsb = jnp.concatenate(
        [jnp.broadcast_to(weight[:, None], (3, L)),
         jnp.broadcast_to(bias[:, None], (1, L))], axis=0).reshape(-1)
    out_flat = _fused_kernel(
        attr_t, residues, wsb, anc_t, pixel_to_node.astype(jnp.int32))
    return out_flat.reshape(Bn, Cn, Hn, Wn)
